# Initial kernel scaffold; baseline (speedup 1.0000x reference)
import jax
import jax.numpy as jnp
from jax.experimental import pallas as pl


def kernel(x, edge_index, inferenz_anchors, W1, b1, W2, b2):
    raise NotImplementedError("write your pallas kernel here")


# speedup vs baseline: 7.9140x; 7.9140x over previous
import functools

import jax
import jax.numpy as jnp
from jax import lax
from jax.experimental import pallas as pl
from jax.experimental.pallas import tpu as pltpu
from jax.experimental.pallas import tpu_sc as plsc

N = 10000
E = 320000
D = 128
NPAD = 10240
DUMMY = N
NC, NS = 2, 16
NW = NC * NS
CHUNK = 128
CPT = 80
G = 16
EPT = CPT * CHUNK
EPAD = NW * EPT
DEGW = 16
RPT = NPAD // NS
RB = 1280

@functools.cache
def _sc_kernels():
    mesh = plsc.VectorSubcoreMesh(
        core_axis_name="c", subcore_axis_name="s", num_cores=NC, num_subcores=NS)
    deg = _make_deg(mesh)
    spmv = _make_spmv(mesh)
    return deg, spmv


def _make_deg(mesh):
  @functools.partial(
    pl.kernel,
    out_type=jax.ShapeDtypeStruct((2 * NPAD, DEGW), jnp.float32),
    mesh=mesh,
    scratch_types=[
        pltpu.VMEM((CPT, CHUNK), jnp.int32),
        pltpu.VMEM((CHUNK, DEGW), jnp.float32),
        pltpu.VMEM((CHUNK, DEGW), jnp.float32),
        pltpu.VMEM_SHARED((NPAD, DEGW), jnp.float32),
    ],
  )
  def _deg_kernel(dstI, ones16, zeros16, out, dst_v, ones_v, zero_v, acc):
        c = lax.axis_index("c")
        s = lax.axis_index("s")
        wid = s * NC + c
        base = s * RPT
        pltpu.sync_copy(zeros16, zero_v)
        for k in range(RPT // CHUNK):
            pltpu.sync_copy(zero_v, acc.at[pl.ds(base + k * CHUNK, CHUNK)])
        pltpu.sync_copy(ones16, ones_v)
        pltpu.sync_copy(dstI.at[wid], dst_v)
        plsc.subcore_barrier()

        def body(j, carry):
            pltpu.sync_copy(ones_v, acc.at[dst_v.at[j]], add=True)
            return carry

        lax.fori_loop(0, CPT, body, 0)
        plsc.subcore_barrier()
        for k in range(RPT // CHUNK):
            pltpu.sync_copy(acc.at[pl.ds(base + k * CHUNK, CHUNK)], ones_v)
            pltpu.sync_copy(ones_v, out.at[pl.ds(c * NPAD + base + k * CHUNK, CHUNK)])

  return _deg_kernel


def _make_spmv(mesh):
  @functools.partial(
    pl.kernel,
    out_type=jax.ShapeDtypeStruct((2 * NPAD, D), jnp.float32),
    mesh=mesh,
    scratch_types=[
        pltpu.VMEM((G, CHUNK), jnp.int32),
        pltpu.VMEM((G, CHUNK), jnp.int32),
        pltpu.VMEM((CHUNK, D), jnp.float32),
        pltpu.VMEM((CHUNK, D), jnp.float32),
        pltpu.SemaphoreType.DMA,
        pltpu.SemaphoreType.DMA,
        pltpu.VMEM_SHARED((NPAD, D), jnp.float32),
    ],
  )
  def _spmv_kernel(hs, srcI, dstI, zeros128, out, src_v, dst_v, rb0, rb1, sem0, sem1, acc):
        c = lax.axis_index("c")
        s = lax.axis_index("s")
        wid = s * NC + c
        base = s * RPT
        pltpu.sync_copy(zeros128, rb0)
        for k in range(RPT // CHUNK):
            pltpu.sync_copy(rb0, acc.at[pl.ds(base + k * CHUNK, CHUNK)])
        plsc.subcore_barrier()

        def super_chunk(g, carry):
            pltpu.sync_copy(srcI.at[wid].at[pl.ds(g * G, G)], src_v)
            pltpu.sync_copy(dstI.at[wid].at[pl.ds(g * G, G)], dst_v)
            pltpu.async_copy(hs.at[src_v.at[0]], rb0, sem0)

            def body(i, carry2):
                j0 = i * 2
                pltpu.make_async_copy(hs.at[src_v.at[j0]], rb0, sem0).wait()
                pltpu.async_copy(hs.at[src_v.at[j0 + 1]], rb1, sem1)
                pltpu.sync_copy(rb0, acc.at[dst_v.at[j0]], add=True)
                pltpu.make_async_copy(hs.at[src_v.at[j0 + 1]], rb1, sem1).wait()

                @pl.when(j0 + 2 < G)
                def _():
                    pltpu.async_copy(hs.at[src_v.at[j0 + 2]], rb0, sem0)

                pltpu.sync_copy(rb1, acc.at[dst_v.at[j0 + 1]], add=True)
                return carry2

            lax.fori_loop(0, G // 2, body, 0)
            return carry

        lax.fori_loop(0, CPT // G, super_chunk, 0)
        plsc.subcore_barrier()
        for k in range(RPT // CHUNK):
            pltpu.sync_copy(acc.at[pl.ds(base + k * CHUNK, CHUNK)], rb0)
            pltpu.sync_copy(rb0, out.at[pl.ds(c * NPAD + base + k * CHUNK, CHUNK)])

  return _spmv_kernel


def _t1_body(x_ref, a_ref, w1a_ref, w1b_ref, dp0_ref, dp1_ref, hs1_ref, dis_ref):
    xa = x_ref[...] - a_ref[...]
    h1 = jnp.dot(xa, w1a_ref[...], preferred_element_type=jnp.float32)
    h1 = h1 + jnp.dot(a_ref[...], w1b_ref[...], preferred_element_type=jnp.float32)
    deg = dp0_ref[...] + dp1_ref[...] + 1.0
    dis = lax.rsqrt(deg)
    dis_ref[...] = dis
    hs1_ref[...] = h1 * dis[:, 0:1]


def _t1(xp, ap, w1a, w1b, dp0, dp1):
    return pl.pallas_call(
        _t1_body,
        grid=(NPAD // RB,),
        in_specs=[
            pl.BlockSpec((RB, D), lambda i: (i, 0)),
            pl.BlockSpec((RB, D), lambda i: (i, 0)),
            pl.BlockSpec((D, D), lambda i: (0, 0)),
            pl.BlockSpec((D, D), lambda i: (0, 0)),
            pl.BlockSpec((RB, DEGW), lambda i: (i, 0)),
            pl.BlockSpec((RB, DEGW), lambda i: (i, 0)),
        ],
        out_specs=[
            pl.BlockSpec((RB, D), lambda i: (i, 0)),
            pl.BlockSpec((RB, DEGW), lambda i: (i, 0)),
        ],
        out_shape=[
            jax.ShapeDtypeStruct((NPAD, D), jnp.float32),
            jax.ShapeDtypeStruct((NPAD, DEGW), jnp.float32),
        ],
    )(xp, ap, w1a, w1b, dp0, dp1)


def _t2_body(p0_ref, p1_ref, hs1_ref, dis_ref, w2_ref, b1_ref, hs2_ref):
    d = dis_ref[...][:, 0:1]
    agg = d * (p0_ref[...] + p1_ref[...] + hs1_ref[...]) + b1_ref[...]
    x2 = jnp.maximum(agg, 0.0)
    h2 = jnp.dot(x2, w2_ref[...], preferred_element_type=jnp.float32)
    hs2_ref[...] = h2 * d


def _t2(p0, p1, hs1, dis, w2, b1):
    return pl.pallas_call(
        _t2_body,
        grid=(NPAD // RB,),
        in_specs=[
            pl.BlockSpec((RB, D), lambda i: (i, 0)),
            pl.BlockSpec((RB, D), lambda i: (i, 0)),
            pl.BlockSpec((RB, D), lambda i: (i, 0)),
            pl.BlockSpec((RB, DEGW), lambda i: (i, 0)),
            pl.BlockSpec((D, D), lambda i: (0, 0)),
            pl.BlockSpec((1, D), lambda i: (0, 0)),
        ],
        out_specs=pl.BlockSpec((RB, D), lambda i: (i, 0)),
        out_shape=jax.ShapeDtypeStruct((NPAD, D), jnp.float32),
    )(p0, p1, hs1, dis, w2, b1)


def _t3_body(p0_ref, p1_ref, hs2_ref, dis_ref, b2_ref, out_ref):
    d = dis_ref[...][:, 0:1]
    out_ref[...] = d * (p0_ref[...] + p1_ref[...] + hs2_ref[...]) + b2_ref[...]


def _t3(p0, p1, hs2, dis, b2):
    return pl.pallas_call(
        _t3_body,
        grid=(NPAD // RB,),
        in_specs=[
            pl.BlockSpec((RB, D), lambda i: (i, 0)),
            pl.BlockSpec((RB, D), lambda i: (i, 0)),
            pl.BlockSpec((RB, D), lambda i: (i, 0)),
            pl.BlockSpec((RB, DEGW), lambda i: (i, 0)),
            pl.BlockSpec((1, D), lambda i: (0, 0)),
        ],
        out_specs=pl.BlockSpec((RB, D), lambda i: (i, 0)),
        out_shape=jax.ShapeDtypeStruct((NPAD, D), jnp.float32),
    )(p0, p1, hs2, dis, b2)


def kernel(x, edge_index, inferenz_anchors, W1, b1, W2, b2):
    src = edge_index[0].astype(jnp.int32)
    dst = edge_index[1].astype(jnp.int32)
    src_p = jnp.concatenate(
        [src, jnp.zeros((EPAD - E,), jnp.int32)]).reshape(NW, CPT, CHUNK)
    dst_p = jnp.concatenate(
        [dst, jnp.full((EPAD - E,), DUMMY, jnp.int32)]).reshape(NW, CPT, CHUNK)
    xp = jnp.zeros((NPAD, D), jnp.float32).at[:N].set(x)
    ap = jnp.zeros((NPAD, D), jnp.float32).at[:N].set(inferenz_anchors)
    zeros128 = jnp.zeros((CHUNK, D), jnp.float32)
    ones16 = jnp.ones((CHUNK, DEGW), jnp.float32)
    zeros16 = jnp.zeros((CHUNK, DEGW), jnp.float32)
    w1a, w1b = W1[:D], W1[D:]

    deg_kernel, spmv_kernel = _sc_kernels()
    degp = deg_kernel(dst_p, ones16, zeros16)
    hs1, dis = _t1(xp, ap, w1a, w1b, degp[:NPAD], degp[NPAD:])
    p1 = spmv_kernel(hs1, src_p, dst_p, zeros128)
    hs2 = _t2(p1[:NPAD], p1[NPAD:], hs1, dis, W2, b1.reshape(1, D))
    p2 = spmv_kernel(hs2, src_p, dst_p, zeros128)
    outp = _t3(p2[:NPAD], p2[NPAD:], hs2, dis, b2.reshape(1, D))
    return outp[:N]


# speedup vs baseline: 9.0052x; 1.1379x over previous
import functools

import jax
import jax.numpy as jnp
from jax import lax
from jax.experimental import pallas as pl
from jax.experimental.pallas import tpu as pltpu
from jax.experimental.pallas import tpu_sc as plsc

N = 10000
E = 320000
D = 128
NPAD = 10240
DUMMY = N
NC, NS = 2, 16
NW = NC * NS
CHUNK = 128
CPT = 80
G = 16
EPT = CPT * CHUNK
EPAD = NW * EPT
DEGW = 16
RPT = NPAD // NS
RB = 1280

@functools.cache
def _sc_kernels():
    mesh = plsc.VectorSubcoreMesh(
        core_axis_name="c", subcore_axis_name="s", num_cores=NC, num_subcores=NS)
    deg = _make_deg(mesh)
    spmv = _make_spmv(mesh)
    return deg, spmv


def _make_deg(mesh):
  @functools.partial(
    pl.kernel,
    out_type=jax.ShapeDtypeStruct((2 * NPAD, D), jnp.float32),
    mesh=mesh,
    scratch_types=[
        pltpu.VMEM((G, CHUNK), jnp.int32),
        pltpu.VMEM((CHUNK, D), jnp.float32),
        pltpu.VMEM_SHARED((NPAD, D), jnp.float32),
    ],
  )
  def _deg_kernel(dstI, ones128, zeros128, out, dst_v, buf, acc):
        c = lax.axis_index("c")
        s = lax.axis_index("s")
        wid = s * NC + c
        base = s * RPT
        pltpu.sync_copy(zeros128, buf)
        for k in range(RPT // CHUNK):
            pltpu.sync_copy(buf, acc.at[pl.ds(base + k * CHUNK, CHUNK)])
        pltpu.sync_copy(ones128, buf)
        plsc.subcore_barrier()

        def super_chunk(g, carry):
            pltpu.sync_copy(dstI.at[wid].at[pl.ds(g * G, G)], dst_v)

            def body(j, carry2):
                pltpu.sync_copy(buf, acc.at[dst_v.at[j]], add=True)
                return carry2

            lax.fori_loop(0, G, body, 0)
            return carry

        lax.fori_loop(0, CPT // G, super_chunk, 0)
        plsc.subcore_barrier()
        for k in range(RPT // CHUNK):
            pltpu.sync_copy(acc.at[pl.ds(base + k * CHUNK, CHUNK)], buf)
            pltpu.sync_copy(buf, out.at[pl.ds(c * NPAD + base + k * CHUNK, CHUNK)])

  return _deg_kernel


def _make_spmv(mesh):
  @functools.partial(
    pl.kernel,
    out_type=jax.ShapeDtypeStruct((2 * NPAD, D), jnp.float32),
    mesh=mesh,
    scratch_types=[
        pltpu.VMEM((G, CHUNK), jnp.int32),
        pltpu.VMEM((G, CHUNK), jnp.int32),
        pltpu.VMEM((CHUNK, D), jnp.float32),
        pltpu.VMEM((CHUNK, D), jnp.float32),
        pltpu.SemaphoreType.DMA,
        pltpu.SemaphoreType.DMA,
        pltpu.VMEM_SHARED((NPAD, D), jnp.float32),
    ],
  )
  def _spmv_kernel(hs, srcI, dstI, zeros128, out, src_v, dst_v, rb0, rb1, sem0, sem1, acc):
        c = lax.axis_index("c")
        s = lax.axis_index("s")
        wid = s * NC + c
        base = s * RPT
        pltpu.sync_copy(zeros128, rb0)
        for k in range(RPT // CHUNK):
            pltpu.sync_copy(rb0, acc.at[pl.ds(base + k * CHUNK, CHUNK)])
        plsc.subcore_barrier()

        def super_chunk(g, carry):
            pltpu.sync_copy(srcI.at[wid].at[pl.ds(g * G, G)], src_v)
            pltpu.sync_copy(dstI.at[wid].at[pl.ds(g * G, G)], dst_v)
            pltpu.async_copy(hs.at[src_v.at[0]], rb0, sem0)

            def body(i, carry2):
                j0 = i * 2
                pltpu.make_async_copy(hs.at[src_v.at[j0]], rb0, sem0).wait()
                pltpu.async_copy(hs.at[src_v.at[j0 + 1]], rb1, sem1)
                pltpu.sync_copy(rb0, acc.at[dst_v.at[j0]], add=True)
                pltpu.make_async_copy(hs.at[src_v.at[j0 + 1]], rb1, sem1).wait()

                @pl.when(j0 + 2 < G)
                def _():
                    pltpu.async_copy(hs.at[src_v.at[j0 + 2]], rb0, sem0)

                pltpu.sync_copy(rb1, acc.at[dst_v.at[j0 + 1]], add=True)
                return carry2

            lax.fori_loop(0, G // 2, body, 0)
            return carry

        lax.fori_loop(0, CPT // G, super_chunk, 0)
        plsc.subcore_barrier()
        for k in range(RPT // CHUNK):
            pltpu.sync_copy(acc.at[pl.ds(base + k * CHUNK, CHUNK)], rb0)
            pltpu.sync_copy(rb0, out.at[pl.ds(c * NPAD + base + k * CHUNK, CHUNK)])

  return _spmv_kernel


def _t1_body(x_ref, a_ref, w1a_ref, w1b_ref, dp0_ref, dp1_ref, hs1_ref, dis_ref):
    xa = x_ref[...] - a_ref[...]
    h1 = jnp.dot(xa, w1a_ref[...], preferred_element_type=jnp.float32)
    h1 = h1 + jnp.dot(a_ref[...], w1b_ref[...], preferred_element_type=jnp.float32)
    deg = dp0_ref[...] + dp1_ref[...] + 1.0
    dis = lax.rsqrt(deg)
    dis_ref[...] = dis
    hs1_ref[...] = h1 * dis


def _t1(xp, ap, w1a, w1b, dp0, dp1):
    return pl.pallas_call(
        _t1_body,
        grid=(NPAD // RB,),
        in_specs=[
            pl.BlockSpec((RB, D), lambda i: (i, 0)),
            pl.BlockSpec((RB, D), lambda i: (i, 0)),
            pl.BlockSpec((D, D), lambda i: (0, 0)),
            pl.BlockSpec((D, D), lambda i: (0, 0)),
            pl.BlockSpec((RB, D), lambda i: (i, 0)),
            pl.BlockSpec((RB, D), lambda i: (i, 0)),
        ],
        out_specs=[
            pl.BlockSpec((RB, D), lambda i: (i, 0)),
            pl.BlockSpec((RB, D), lambda i: (i, 0)),
        ],
        out_shape=[
            jax.ShapeDtypeStruct((NPAD, D), jnp.float32),
            jax.ShapeDtypeStruct((NPAD, D), jnp.float32),
        ],
    )(xp, ap, w1a, w1b, dp0, dp1)


def _t2_body(p0_ref, p1_ref, hs1_ref, dis_ref, w2_ref, b1_ref, hs2_ref):
    d = dis_ref[...]
    agg = d * (p0_ref[...] + p1_ref[...] + hs1_ref[...]) + b1_ref[...]
    x2 = jnp.maximum(agg, 0.0)
    h2 = jnp.dot(x2, w2_ref[...], preferred_element_type=jnp.float32)
    hs2_ref[...] = h2 * d


def _t2(p0, p1, hs1, dis, w2, b1):
    return pl.pallas_call(
        _t2_body,
        grid=(NPAD // RB,),
        in_specs=[
            pl.BlockSpec((RB, D), lambda i: (i, 0)),
            pl.BlockSpec((RB, D), lambda i: (i, 0)),
            pl.BlockSpec((RB, D), lambda i: (i, 0)),
            pl.BlockSpec((RB, D), lambda i: (i, 0)),
            pl.BlockSpec((D, D), lambda i: (0, 0)),
            pl.BlockSpec((1, D), lambda i: (0, 0)),
        ],
        out_specs=pl.BlockSpec((RB, D), lambda i: (i, 0)),
        out_shape=jax.ShapeDtypeStruct((NPAD, D), jnp.float32),
    )(p0, p1, hs1, dis, w2, b1)


def _t3_body(p0_ref, p1_ref, hs2_ref, dis_ref, b2_ref, out_ref):
    d = dis_ref[...]
    out_ref[...] = d * (p0_ref[...] + p1_ref[...] + hs2_ref[...]) + b2_ref[...]


def _t3(p0, p1, hs2, dis, b2):
    return pl.pallas_call(
        _t3_body,
        grid=(NPAD // RB,),
        in_specs=[
            pl.BlockSpec((RB, D), lambda i: (i, 0)),
            pl.BlockSpec((RB, D), lambda i: (i, 0)),
            pl.BlockSpec((RB, D), lambda i: (i, 0)),
            pl.BlockSpec((RB, D), lambda i: (i, 0)),
            pl.BlockSpec((1, D), lambda i: (0, 0)),
        ],
        out_specs=pl.BlockSpec((RB, D), lambda i: (i, 0)),
        out_shape=jax.ShapeDtypeStruct((NPAD, D), jnp.float32),
    )(p0, p1, hs2, dis, b2)


def kernel(x, edge_index, inferenz_anchors, W1, b1, W2, b2):
    src = edge_index[0].astype(jnp.int32)
    dst = edge_index[1].astype(jnp.int32)
    ept_real = E // NW
    npad_e = EPT - ept_real
    pad_src = jnp.zeros((NW, npad_e), jnp.int32)
    pad_dst = jnp.broadcast_to(
        jnp.arange(DUMMY, DUMMY + npad_e, dtype=jnp.int32)[None], (NW, npad_e))
    src_p = jnp.concatenate(
        [src.reshape(NW, ept_real), pad_src], axis=1).reshape(NW, CPT, CHUNK)
    dst_p = jnp.concatenate(
        [dst.reshape(NW, ept_real), pad_dst], axis=1).reshape(NW, CPT, CHUNK)
    xp = jnp.zeros((NPAD, D), jnp.float32).at[:N].set(x)
    ap = jnp.zeros((NPAD, D), jnp.float32).at[:N].set(inferenz_anchors)
    zeros128 = jnp.zeros((CHUNK, D), jnp.float32)
    ones128 = jnp.ones((CHUNK, D), jnp.float32)
    w1a, w1b = W1[:D], W1[D:]

    deg_kernel, spmv_kernel = _sc_kernels()
    degp = deg_kernel(dst_p, ones128, zeros128)
    hs1, dis = _t1(xp, ap, w1a, w1b, degp[:NPAD], degp[NPAD:])
    p1 = spmv_kernel(hs1, src_p, dst_p, zeros128)
    hs2 = _t2(p1[:NPAD], p1[NPAD:], hs1, dis, W2, b1.reshape(1, D))
    p2 = spmv_kernel(hs2, src_p, dst_p, zeros128)
    outp = _t3(p2[:NPAD], p2[NPAD:], hs2, dis, b2.reshape(1, D))
    return outp[:N]


# speedup vs baseline: 9.0157x; 1.0012x over previous
import functools

import jax
import jax.numpy as jnp
from jax import lax
from jax.experimental import pallas as pl
from jax.experimental.pallas import tpu as pltpu
from jax.experimental.pallas import tpu_sc as plsc

N = 10000
E = 320000
D = 128
NPAD = 10240
DUMMY = N
NC, NS = 2, 16
NW = NC * NS
CHUNK = 128
CPT = 80
G = 16
EPT = CPT * CHUNK
EPAD = NW * EPT
DEGW = 16
RPT = NPAD // NS
RB = 1280

@functools.cache
def _sc_kernels():
    mesh = plsc.VectorSubcoreMesh(
        core_axis_name="c", subcore_axis_name="s", num_cores=NC, num_subcores=NS)
    deg = _make_deg(mesh)
    spmv = _make_spmv(mesh)
    return deg, spmv


def _make_deg(mesh):
  @functools.partial(
    pl.kernel,
    out_type=jax.ShapeDtypeStruct((2 * NPAD, D), jnp.float32),
    mesh=mesh,
    scratch_types=[
        pltpu.VMEM((G, CHUNK), jnp.int32),
        pltpu.VMEM((CHUNK, D), jnp.float32),
        pltpu.VMEM_SHARED((NPAD, D), jnp.float32),
    ],
  )
  def _deg_kernel(dstI, ones128, zeros128, out, dst_v, buf, acc):
        c = lax.axis_index("c")
        s = lax.axis_index("s")
        wid = s * NC + c
        base = s * RPT
        pltpu.sync_copy(zeros128, buf)
        for k in range(RPT // CHUNK):
            pltpu.sync_copy(buf, acc.at[pl.ds(base + k * CHUNK, CHUNK)])
        pltpu.sync_copy(ones128, buf)
        plsc.subcore_barrier()

        def super_chunk(g, carry):
            pltpu.sync_copy(dstI.at[wid].at[pl.ds(g * G, G)], dst_v)

            def body(j, carry2):
                pltpu.sync_copy(buf, acc.at[dst_v.at[j]], add=True)
                return carry2

            lax.fori_loop(0, G, body, 0)
            return carry

        lax.fori_loop(0, CPT // G, super_chunk, 0)
        plsc.subcore_barrier()
        for k in range(RPT // CHUNK):
            pltpu.sync_copy(acc.at[pl.ds(base + k * CHUNK, CHUNK)], buf)
            pltpu.sync_copy(buf, out.at[pl.ds(c * NPAD + base + k * CHUNK, CHUNK)])

  return _deg_kernel


def _make_spmv(mesh):
  @functools.partial(
    pl.kernel,
    out_type=jax.ShapeDtypeStruct((2 * NPAD, D), jnp.float32),
    mesh=mesh,
    scratch_types=[
        pltpu.VMEM((G, CHUNK), jnp.int32),
        pltpu.VMEM((G, CHUNK), jnp.int32),
        pltpu.VMEM((CHUNK, D), jnp.float32),
        pltpu.VMEM((CHUNK, D), jnp.float32),
        pltpu.SemaphoreType.DMA,
        pltpu.SemaphoreType.DMA,
        pltpu.SemaphoreType.DMA,
        pltpu.SemaphoreType.DMA,
        pltpu.VMEM_SHARED((NPAD, D), jnp.float32),
    ],
  )
  def _spmv_kernel(hs, srcI, dstI, zeros128, out, src_v, dst_v, rb0, rb1,
                   semg0, semg1, sems0, sems1, acc):
        c = lax.axis_index("c")
        s = lax.axis_index("s")
        wid = s * NC + c
        base = s * RPT
        pltpu.sync_copy(zeros128, rb0)
        for k in range(RPT // CHUNK):
            pltpu.sync_copy(rb0, acc.at[pl.ds(base + k * CHUNK, CHUNK)])
        plsc.subcore_barrier()

        def super_chunk(g, carry):
            pltpu.sync_copy(srcI.at[wid].at[pl.ds(g * G, G)], src_v)
            pltpu.sync_copy(dstI.at[wid].at[pl.ds(g * G, G)], dst_v)
            pltpu.async_copy(hs.at[src_v.at[0]], rb0, semg0)

            def body(i, carry2):
                j0 = i * 2
                pltpu.make_async_copy(hs.at[src_v.at[j0]], rb0, semg0).wait()

                @pl.when(i > 0)
                def _():
                    pltpu.make_async_copy(rb1, acc.at[dst_v.at[j0 - 1]], sems1).wait()

                pltpu.async_copy(hs.at[src_v.at[j0 + 1]], rb1, semg1)
                pltpu.async_copy(rb0, acc.at[dst_v.at[j0]], sems0, add=True)
                pltpu.make_async_copy(hs.at[src_v.at[j0 + 1]], rb1, semg1).wait()
                pltpu.make_async_copy(rb0, acc.at[dst_v.at[j0]], sems0).wait()

                @pl.when(j0 + 2 < G)
                def _():
                    pltpu.async_copy(hs.at[src_v.at[j0 + 2]], rb0, semg0)

                pltpu.async_copy(rb1, acc.at[dst_v.at[j0 + 1]], sems1, add=True)
                return carry2

            lax.fori_loop(0, G // 2, body, 0)
            pltpu.make_async_copy(rb1, acc.at[dst_v.at[G - 1]], sems1).wait()
            return carry

        lax.fori_loop(0, CPT // G, super_chunk, 0)
        plsc.subcore_barrier()
        for k in range(RPT // CHUNK):
            pltpu.sync_copy(acc.at[pl.ds(base + k * CHUNK, CHUNK)], rb0)
            pltpu.sync_copy(rb0, out.at[pl.ds(c * NPAD + base + k * CHUNK, CHUNK)])

  return _spmv_kernel


def _t1_body(x_ref, a_ref, w1a_ref, w1b_ref, dp0_ref, dp1_ref, hs1_ref, dis_ref):
    xa = x_ref[...] - a_ref[...]
    h1 = jnp.dot(xa, w1a_ref[...], preferred_element_type=jnp.float32)
    h1 = h1 + jnp.dot(a_ref[...], w1b_ref[...], preferred_element_type=jnp.float32)
    deg = dp0_ref[...] + dp1_ref[...] + 1.0
    dis = lax.rsqrt(deg)
    dis_ref[...] = dis
    hs1_ref[...] = h1 * dis


def _t1(xp, ap, w1a, w1b, dp0, dp1):
    return pl.pallas_call(
        _t1_body,
        grid=(NPAD // RB,),
        in_specs=[
            pl.BlockSpec((RB, D), lambda i: (i, 0)),
            pl.BlockSpec((RB, D), lambda i: (i, 0)),
            pl.BlockSpec((D, D), lambda i: (0, 0)),
            pl.BlockSpec((D, D), lambda i: (0, 0)),
            pl.BlockSpec((RB, D), lambda i: (i, 0)),
            pl.BlockSpec((RB, D), lambda i: (i, 0)),
        ],
        out_specs=[
            pl.BlockSpec((RB, D), lambda i: (i, 0)),
            pl.BlockSpec((RB, D), lambda i: (i, 0)),
        ],
        out_shape=[
            jax.ShapeDtypeStruct((NPAD, D), jnp.float32),
            jax.ShapeDtypeStruct((NPAD, D), jnp.float32),
        ],
    )(xp, ap, w1a, w1b, dp0, dp1)


def _t2_body(p0_ref, p1_ref, hs1_ref, dis_ref, w2_ref, b1_ref, hs2_ref):
    d = dis_ref[...]
    agg = d * (p0_ref[...] + p1_ref[...] + hs1_ref[...]) + b1_ref[...]
    x2 = jnp.maximum(agg, 0.0)
    h2 = jnp.dot(x2, w2_ref[...], preferred_element_type=jnp.float32)
    hs2_ref[...] = h2 * d


def _t2(p0, p1, hs1, dis, w2, b1):
    return pl.pallas_call(
        _t2_body,
        grid=(NPAD // RB,),
        in_specs=[
            pl.BlockSpec((RB, D), lambda i: (i, 0)),
            pl.BlockSpec((RB, D), lambda i: (i, 0)),
            pl.BlockSpec((RB, D), lambda i: (i, 0)),
            pl.BlockSpec((RB, D), lambda i: (i, 0)),
            pl.BlockSpec((D, D), lambda i: (0, 0)),
            pl.BlockSpec((1, D), lambda i: (0, 0)),
        ],
        out_specs=pl.BlockSpec((RB, D), lambda i: (i, 0)),
        out_shape=jax.ShapeDtypeStruct((NPAD, D), jnp.float32),
    )(p0, p1, hs1, dis, w2, b1)


def _t3_body(p0_ref, p1_ref, hs2_ref, dis_ref, b2_ref, out_ref):
    d = dis_ref[...]
    out_ref[...] = d * (p0_ref[...] + p1_ref[...] + hs2_ref[...]) + b2_ref[...]


def _t3(p0, p1, hs2, dis, b2):
    return pl.pallas_call(
        _t3_body,
        grid=(NPAD // RB,),
        in_specs=[
            pl.BlockSpec((RB, D), lambda i: (i, 0)),
            pl.BlockSpec((RB, D), lambda i: (i, 0)),
            pl.BlockSpec((RB, D), lambda i: (i, 0)),
            pl.BlockSpec((RB, D), lambda i: (i, 0)),
            pl.BlockSpec((1, D), lambda i: (0, 0)),
        ],
        out_specs=pl.BlockSpec((RB, D), lambda i: (i, 0)),
        out_shape=jax.ShapeDtypeStruct((NPAD, D), jnp.float32),
    )(p0, p1, hs2, dis, b2)


def kernel(x, edge_index, inferenz_anchors, W1, b1, W2, b2):
    src = edge_index[0].astype(jnp.int32)
    dst = edge_index[1].astype(jnp.int32)
    ept_real = E // NW
    npad_e = EPT - ept_real
    pad_src = jnp.zeros((NW, npad_e), jnp.int32)
    pad_dst = jnp.broadcast_to(
        jnp.arange(DUMMY, DUMMY + npad_e, dtype=jnp.int32)[None], (NW, npad_e))
    src_p = jnp.concatenate(
        [src.reshape(NW, ept_real), pad_src], axis=1).reshape(NW, CPT, CHUNK)
    dst_p = jnp.concatenate(
        [dst.reshape(NW, ept_real), pad_dst], axis=1).reshape(NW, CPT, CHUNK)
    xp = jnp.zeros((NPAD, D), jnp.float32).at[:N].set(x)
    ap = jnp.zeros((NPAD, D), jnp.float32).at[:N].set(inferenz_anchors)
    zeros128 = jnp.zeros((CHUNK, D), jnp.float32)
    ones128 = jnp.ones((CHUNK, D), jnp.float32)
    w1a, w1b = W1[:D], W1[D:]

    deg_kernel, spmv_kernel = _sc_kernels()
    degp = deg_kernel(dst_p, ones128, zeros128)
    hs1, dis = _t1(xp, ap, w1a, w1b, degp[:NPAD], degp[NPAD:])
    p1 = spmv_kernel(hs1, src_p, dst_p, zeros128)
    hs2 = _t2(p1[:NPAD], p1[NPAD:], hs1, dis, W2, b1.reshape(1, D))
    p2 = spmv_kernel(hs2, src_p, dst_p, zeros128)
    outp = _t3(p2[:NPAD], p2[NPAD:], hs2, dis, b2.reshape(1, D))
    return outp[:N]


# speedup vs baseline: 9.1129x; 1.0108x over previous
import functools

import jax
import jax.numpy as jnp
from jax import lax
from jax.experimental import pallas as pl
from jax.experimental.pallas import tpu as pltpu
from jax.experimental.pallas import tpu_sc as plsc

N = 10000
E = 320000
D = 128
NPAD = 10240
DUMMY = N
NC, NS = 2, 16
NW = NC * NS
CHUNK = 128
CPT = 80
G = 40
EPT = CPT * CHUNK
EPAD = NW * EPT
DEGW = 16
RPT = NPAD // NS
RB = 1280

@functools.cache
def _sc_kernels():
    mesh = plsc.VectorSubcoreMesh(
        core_axis_name="c", subcore_axis_name="s", num_cores=NC, num_subcores=NS)
    deg = _make_deg(mesh)
    spmv = _make_spmv(mesh)
    return deg, spmv


def _make_deg(mesh):
  @functools.partial(
    pl.kernel,
    out_type=jax.ShapeDtypeStruct((2 * NPAD, D), jnp.float32),
    mesh=mesh,
    scratch_types=[
        pltpu.VMEM((G, CHUNK), jnp.int32),
        pltpu.VMEM((CHUNK, D), jnp.float32),
        pltpu.VMEM_SHARED((NPAD, D), jnp.float32),
    ],
  )
  def _deg_kernel(dstI, ones128, zeros128, out, dst_v, buf, acc):
        c = lax.axis_index("c")
        s = lax.axis_index("s")
        wid = s * NC + c
        base = s * RPT
        pltpu.sync_copy(zeros128, buf)
        for k in range(RPT // CHUNK):
            pltpu.sync_copy(buf, acc.at[pl.ds(base + k * CHUNK, CHUNK)])
        pltpu.sync_copy(ones128, buf)
        plsc.subcore_barrier()

        def super_chunk(g, carry):
            pltpu.sync_copy(dstI.at[wid].at[pl.ds(g * G, G)], dst_v)

            def body(j, carry2):
                pltpu.sync_copy(buf, acc.at[dst_v.at[j]], add=True)
                return carry2

            lax.fori_loop(0, G, body, 0)
            return carry

        lax.fori_loop(0, CPT // G, super_chunk, 0)
        plsc.subcore_barrier()
        for k in range(RPT // CHUNK):
            pltpu.sync_copy(acc.at[pl.ds(base + k * CHUNK, CHUNK)], buf)
            pltpu.sync_copy(buf, out.at[pl.ds(c * NPAD + base + k * CHUNK, CHUNK)])

  return _deg_kernel


def _make_spmv(mesh):
  @functools.partial(
    pl.kernel,
    out_type=jax.ShapeDtypeStruct((2 * NPAD, D), jnp.float32),
    mesh=mesh,
    scratch_types=[
        pltpu.VMEM((G, CHUNK), jnp.int32),
        pltpu.VMEM((G, CHUNK), jnp.int32),
        pltpu.VMEM((CHUNK, D), jnp.float32),
        pltpu.VMEM((CHUNK, D), jnp.float32),
        pltpu.SemaphoreType.DMA,
        pltpu.SemaphoreType.DMA,
        pltpu.SemaphoreType.DMA,
        pltpu.SemaphoreType.DMA,
        pltpu.VMEM_SHARED((NPAD, D), jnp.float32),
    ],
  )
  def _spmv_kernel(hs, srcI, dstI, zeros128, out, src_v, dst_v, rb0, rb1,
                   semg0, semg1, sems0, sems1, acc):
        c = lax.axis_index("c")
        s = lax.axis_index("s")
        wid = s * NC + c
        base = s * RPT
        pltpu.sync_copy(zeros128, rb0)
        for k in range(RPT // CHUNK):
            pltpu.sync_copy(rb0, acc.at[pl.ds(base + k * CHUNK, CHUNK)])
        plsc.subcore_barrier()

        def super_chunk(g, carry):
            pltpu.sync_copy(srcI.at[wid].at[pl.ds(g * G, G)], src_v)
            pltpu.sync_copy(dstI.at[wid].at[pl.ds(g * G, G)], dst_v)
            pltpu.async_copy(hs.at[src_v.at[0]], rb0, semg0)

            def body(i, carry2):
                j0 = i * 2
                pltpu.make_async_copy(hs.at[src_v.at[j0]], rb0, semg0).wait()

                @pl.when(i > 0)
                def _():
                    pltpu.make_async_copy(rb1, acc.at[dst_v.at[j0 - 1]], sems1).wait()

                pltpu.async_copy(hs.at[src_v.at[j0 + 1]], rb1, semg1)
                pltpu.async_copy(rb0, acc.at[dst_v.at[j0]], sems0, add=True)
                pltpu.make_async_copy(hs.at[src_v.at[j0 + 1]], rb1, semg1).wait()
                pltpu.make_async_copy(rb0, acc.at[dst_v.at[j0]], sems0).wait()

                @pl.when(j0 + 2 < G)
                def _():
                    pltpu.async_copy(hs.at[src_v.at[j0 + 2]], rb0, semg0)

                pltpu.async_copy(rb1, acc.at[dst_v.at[j0 + 1]], sems1, add=True)
                return carry2

            lax.fori_loop(0, G // 2, body, 0)
            pltpu.make_async_copy(rb1, acc.at[dst_v.at[G - 1]], sems1).wait()
            return carry

        lax.fori_loop(0, CPT // G, super_chunk, 0)
        plsc.subcore_barrier()
        for k in range(RPT // CHUNK):
            pltpu.sync_copy(acc.at[pl.ds(base + k * CHUNK, CHUNK)], rb0)
            pltpu.sync_copy(rb0, out.at[pl.ds(c * NPAD + base + k * CHUNK, CHUNK)])

  return _spmv_kernel


def _t1_body(x_ref, a_ref, w1a_ref, w1b_ref, dp0_ref, dp1_ref, hs1_ref, dis_ref):
    xa = x_ref[...] - a_ref[...]
    h1 = jnp.dot(xa, w1a_ref[...], preferred_element_type=jnp.float32)
    h1 = h1 + jnp.dot(a_ref[...], w1b_ref[...], preferred_element_type=jnp.float32)
    deg = dp0_ref[...] + dp1_ref[...] + 1.0
    dis = lax.rsqrt(deg)
    dis_ref[...] = dis
    hs1_ref[...] = h1 * dis


def _t1(xp, ap, w1a, w1b, dp0, dp1):
    return pl.pallas_call(
        _t1_body,
        grid=(NPAD // RB,),
        in_specs=[
            pl.BlockSpec((RB, D), lambda i: (i, 0)),
            pl.BlockSpec((RB, D), lambda i: (i, 0)),
            pl.BlockSpec((D, D), lambda i: (0, 0)),
            pl.BlockSpec((D, D), lambda i: (0, 0)),
            pl.BlockSpec((RB, D), lambda i: (i, 0)),
            pl.BlockSpec((RB, D), lambda i: (i, 0)),
        ],
        out_specs=[
            pl.BlockSpec((RB, D), lambda i: (i, 0)),
            pl.BlockSpec((RB, D), lambda i: (i, 0)),
        ],
        out_shape=[
            jax.ShapeDtypeStruct((NPAD, D), jnp.float32),
            jax.ShapeDtypeStruct((NPAD, D), jnp.float32),
        ],
    )(xp, ap, w1a, w1b, dp0, dp1)


def _t2_body(p0_ref, p1_ref, hs1_ref, dis_ref, w2_ref, b1_ref, hs2_ref):
    d = dis_ref[...]
    agg = d * (p0_ref[...] + p1_ref[...] + hs1_ref[...]) + b1_ref[...]
    x2 = jnp.maximum(agg, 0.0)
    h2 = jnp.dot(x2, w2_ref[...], preferred_element_type=jnp.float32)
    hs2_ref[...] = h2 * d


def _t2(p0, p1, hs1, dis, w2, b1):
    return pl.pallas_call(
        _t2_body,
        grid=(NPAD // RB,),
        in_specs=[
            pl.BlockSpec((RB, D), lambda i: (i, 0)),
            pl.BlockSpec((RB, D), lambda i: (i, 0)),
            pl.BlockSpec((RB, D), lambda i: (i, 0)),
            pl.BlockSpec((RB, D), lambda i: (i, 0)),
            pl.BlockSpec((D, D), lambda i: (0, 0)),
            pl.BlockSpec((1, D), lambda i: (0, 0)),
        ],
        out_specs=pl.BlockSpec((RB, D), lambda i: (i, 0)),
        out_shape=jax.ShapeDtypeStruct((NPAD, D), jnp.float32),
    )(p0, p1, hs1, dis, w2, b1)


def _t3_body(p0_ref, p1_ref, hs2_ref, dis_ref, b2_ref, out_ref):
    d = dis_ref[...]
    out_ref[...] = d * (p0_ref[...] + p1_ref[...] + hs2_ref[...]) + b2_ref[...]


def _t3(p0, p1, hs2, dis, b2):
    return pl.pallas_call(
        _t3_body,
        grid=(NPAD // RB,),
        in_specs=[
            pl.BlockSpec((RB, D), lambda i: (i, 0)),
            pl.BlockSpec((RB, D), lambda i: (i, 0)),
            pl.BlockSpec((RB, D), lambda i: (i, 0)),
            pl.BlockSpec((RB, D), lambda i: (i, 0)),
            pl.BlockSpec((1, D), lambda i: (0, 0)),
        ],
        out_specs=pl.BlockSpec((RB, D), lambda i: (i, 0)),
        out_shape=jax.ShapeDtypeStruct((NPAD, D), jnp.float32),
    )(p0, p1, hs2, dis, b2)


def kernel(x, edge_index, inferenz_anchors, W1, b1, W2, b2):
    src = edge_index[0].astype(jnp.int32)
    dst = edge_index[1].astype(jnp.int32)
    ept_real = E // NW
    npad_e = EPT - ept_real
    pad_src = jnp.zeros((NW, npad_e), jnp.int32)
    pad_dst = jnp.broadcast_to(
        jnp.arange(DUMMY, DUMMY + npad_e, dtype=jnp.int32)[None], (NW, npad_e))
    src_p = jnp.concatenate(
        [src.reshape(NW, ept_real), pad_src], axis=1).reshape(NW, CPT, CHUNK)
    dst_p = jnp.concatenate(
        [dst.reshape(NW, ept_real), pad_dst], axis=1).reshape(NW, CPT, CHUNK)
    xp = jnp.zeros((NPAD, D), jnp.float32).at[:N].set(x)
    ap = jnp.zeros((NPAD, D), jnp.float32).at[:N].set(inferenz_anchors)
    zeros128 = jnp.zeros((CHUNK, D), jnp.float32)
    ones128 = jnp.ones((CHUNK, D), jnp.float32)
    w1a, w1b = W1[:D], W1[D:]

    deg_kernel, spmv_kernel = _sc_kernels()
    degp = deg_kernel(dst_p, ones128, zeros128)
    hs1, dis = _t1(xp, ap, w1a, w1b, degp[:NPAD], degp[NPAD:])
    p1 = spmv_kernel(hs1, src_p, dst_p, zeros128)
    hs2 = _t2(p1[:NPAD], p1[NPAD:], hs1, dis, W2, b1.reshape(1, D))
    p2 = spmv_kernel(hs2, src_p, dst_p, zeros128)
    outp = _t3(p2[:NPAD], p2[NPAD:], hs2, dis, b2.reshape(1, D))
    return outp[:N]
